# baseline (speedup 1.0000x reference)
import functools

import jax
import jax.numpy as jnp
from jax import lax
from jax.experimental import pallas as pl
from jax.experimental.pallas import tpu as pltpu
from jax.experimental.pallas import tpu_sc as plsc

B = 16384
D = 64
OUT_D = 2 * D
L = 16


def kernel(input, input_position, input_table, pos_table):
    idx = input.astype(jnp.int32)
    pos = jnp.asarray(input_position, jnp.int32).reshape((1,))

    info = plsc.get_sparse_core_info()
    nw = info.num_cores * info.num_subcores
    b_per_w = B // nw
    mesh = plsc.VectorSubcoreMesh(core_axis_name="c", subcore_axis_name="s")

    @functools.partial(
        pl.kernel,
        out_type=jax.ShapeDtypeStruct((B, OUT_D), jnp.float32),
        mesh=mesh,
        compiler_params=pltpu.CompilerParams(use_tc_tiling_on_sc=False),
        scratch_types=[
            pltpu.VMEM((b_per_w,), jnp.int32),
            pltpu.VMEM((b_per_w, D), jnp.float32),
            pltpu.VMEM((b_per_w, OUT_D), jnp.float32),
            pltpu.VMEM((1,), jnp.int32),
            pltpu.VMEM((1, D), jnp.float32),
            pltpu.SemaphoreType.DMA,
            pltpu.SemaphoreType.DMA,
        ],
    )
    def sc_kernel(idx_hbm, pos_hbm, table_hbm, ptab_hbm, out_hbm,
                  idx_v, emb_v, out_v, pidx_v, prow_v, sem, psem):
        wid = lax.axis_index("s") * info.num_cores + lax.axis_index("c")
        base = wid * b_per_w

        pltpu.sync_copy(idx_hbm.at[pl.ds(base, b_per_w)], idx_v)
        gather = pltpu.async_copy(table_hbm.at[idx_v], emb_v, sem)

        pltpu.sync_copy(pos_hbm, pidx_v)
        pltpu.async_copy(ptab_hbm.at[pidx_v], prow_v, psem).wait()
        pvs = [prow_v[0, pl.ds(L * j, L)] for j in range(D // L)]

        def fill(i, carry):
            for j in range(D // L):
                out_v[i, pl.ds(D + L * j, L)] = pvs[j]
            return carry
        lax.fori_loop(0, b_per_w, fill, 0)

        gather.wait()

        def copy_emb(i, carry):
            for j in range(D // L):
                out_v[i, pl.ds(L * j, L)] = emb_v[i, pl.ds(L * j, L)]
            return carry
        lax.fori_loop(0, b_per_w, copy_emb, 0)

        pltpu.sync_copy(out_v, out_hbm.at[pl.ds(base, b_per_w)])

    return sc_kernel(idx, pos, input_table, pos_table)


# speedup vs baseline: 1.7303x; 1.7303x over previous
import functools

import jax
import jax.numpy as jnp
from jax import lax
from jax.experimental import pallas as pl
from jax.experimental.pallas import tpu as pltpu
from jax.experimental.pallas import tpu_sc as plsc

B = 16384
D = 64
OUT_D = 2 * D
L = 16


def kernel(input, input_position, input_table, pos_table):
    idx = input.astype(jnp.int32)
    posrow = jnp.concatenate(
        [jnp.take(pos_table, jnp.asarray(input_position), axis=0),
         jnp.zeros((D,), jnp.float32)])

    info = plsc.get_sparse_core_info()
    nw = info.num_cores * info.num_subcores
    b_per_w = B // nw
    mesh = plsc.VectorSubcoreMesh(core_axis_name="c", subcore_axis_name="s")

    @functools.partial(
        pl.kernel,
        out_type=jax.ShapeDtypeStruct((B, OUT_D), jnp.float32),
        mesh=mesh,
        scratch_types=[
            pltpu.VMEM((b_per_w,), jnp.int32),
            pltpu.VMEM((b_per_w, OUT_D), jnp.float32),
            pltpu.VMEM((2 * D,), jnp.float32),
            pltpu.SemaphoreType.DMA,
        ],
    )
    def sc_kernel(idx_hbm, pos_hbm, table_hbm, out_hbm,
                  idx_v, out_v, prow_v, sem):
        wid = lax.axis_index("s") * info.num_cores + lax.axis_index("c")
        base = wid * b_per_w

        pltpu.sync_copy(idx_hbm.at[pl.ds(base, b_per_w)], idx_v)
        pltpu.sync_copy(pos_hbm, prow_v)
        pvs = [prow_v[pl.ds(L * j, L)] for j in range(D // L)]

        def enqueue(k, carry):
            iv = idx_v[pl.ds(k * L, L)]
            for r in range(L):
                row = k * L + r
                pltpu.async_copy(
                    table_hbm.at[iv[r]], out_v.at[row, pl.ds(0, D)], sem)
            return carry
        lax.fori_loop(0, b_per_w // L, enqueue, 0)

        def fill(i, carry):
            for j in range(D // L):
                out_v[i, pl.ds(D + L * j, L)] = pvs[j]
            return carry
        lax.fori_loop(0, b_per_w, fill, 0)

        def drain(i, carry):
            pltpu.make_async_copy(
                table_hbm.at[0], out_v.at[0, pl.ds(0, D)], sem).wait()
            return carry
        lax.fori_loop(0, b_per_w, drain, 0)

        pltpu.sync_copy(out_v, out_hbm.at[pl.ds(base, b_per_w)])

    return sc_kernel(idx, posrow, input_table)
